# baseline (speedup 1.0000x reference)
import functools

import jax
import jax.numpy as jnp
from jax import lax
from jax.experimental import pallas as pl
from jax.experimental.pallas import tpu as pltpu
from jax.experimental.pallas import tpu_sc as plsc

NY, NX, C, N, BS = 496, 432, 64, 40000, 4
NYNX = NY * NX
BASE = BS * NYNX
NTILES = 32
SEG = BASE // NTILES
SEG_V = SEG // 16
N_V = N // 16
CH = 288
CHV = CH // 16
NCH = SEG // CH
CP = 128

_MESH = plsc.VectorSubcoreMesh(core_axis_name="c", subcore_axis_name="s")
_PARAMS = pltpu.CompilerParams(needs_layout_passes=False)


def _wid():
    return lax.axis_index("s") * 2 + lax.axis_index("c")


@functools.partial(
    pl.kernel,
    out_type=jax.ShapeDtypeStruct((BASE,), jnp.int32),
    mesh=_MESH,
    compiler_params=_PARAMS,
    scratch_types=[
        pltpu.VMEM((N,), jnp.int32),
        pltpu.VMEM((SEG,), jnp.int32),
    ],
)
def _build_inv(flat_hbm, inv_hbm, flat_v, inv_v):
    wid = _wid()
    lo = wid * SEG
    pltpu.sync_copy(flat_hbm, flat_v)

    empty = jnp.full((16,), -1, jnp.int32)

    def fill(i, _):
        inv_v[pl.ds(i * 16, 16)] = empty
        return 0

    lax.fori_loop(0, SEG_V, fill, 0)

    lane = lax.iota(jnp.int32, 16)

    def scan(i, _):
        base16 = flat_v[pl.ds(i * 16, 16)]
        loc = base16 - lo
        mask = (loc >= 0) & (loc < SEG)
        loc = jnp.where(mask, loc, 0)
        ids = lane + i * 16
        plsc.store_scatter(inv_v, [loc], ids, mask=mask)
        return 0

    lax.fori_loop(0, N_V, scan, 0)

    pltpu.sync_copy(inv_v, inv_hbm.at[pl.ds(lo, SEG)])


@functools.partial(
    pl.kernel,
    out_type=jax.ShapeDtypeStruct((BS * C * NYNX,), jnp.float32),
    mesh=_MESH,
    compiler_params=_PARAMS,
    scratch_types=[
        pltpu.VMEM((SEG,), jnp.int32),
        pltpu.VMEM((CH, CP), jnp.float32),
        pltpu.VMEM((CH, CP), jnp.float32),
        pltpu.VMEM((C * CH,), jnp.float32),
        pltpu.VMEM((CH,), jnp.int32),
        pltpu.VMEM((CH,), jnp.int32),
        pltpu.SemaphoreType.DMA,
        pltpu.SemaphoreType.DMA,
        pltpu.SemaphoreType.DMA,
    ],
)
def _fill_canvas(vf_hbm, inv_hbm, out_hbm, inv_v, rows0, rows1, stage_v,
                 idx0, idx1, gsem0, gsem1, osem):
    wid = _wid()
    b = wid // 8
    seg_lo = (wid % 8) * SEG
    pltpu.sync_copy(inv_hbm.at[pl.ds(wid * SEG, SEG)], inv_v)

    zero16 = jnp.zeros((16,), jnp.int32)
    lane = lax.iota(jnp.int32, 16)

    def prep_idx(k, idxb):
        def body(j, _):
            iv = inv_v[pl.ds(k * CH + j * 16, 16)]
            idxb[pl.ds(j * 16, 16)] = jnp.maximum(iv, 0)
            return 0

        lax.fori_loop(0, CHV, body, 0)

    def issue_gather(idxb, rowsb, gsem):
        for lo, sz in ((0, 128), (128, 128), (256, 32)):
            pltpu.async_copy(
                vf_hbm.at[idxb.at[pl.ds(lo, sz)]],
                rowsb.at[pl.ds(lo, sz), :],
                gsem,
            )

    def wait_gather(idxb, rowsb, gsem):
        pltpu.make_async_copy(vf_hbm.at[idxb], rowsb, gsem).wait()

    def transpose(k, rowsb):
        def tv(v, _):
            iv = inv_v[pl.ds(k * CH + v * 16, 16)]
            mult = jnp.where(iv >= 0, jnp.float32(1.0), jnp.float32(0.0))
            row16 = lane + v * 16
            for c in range(C):
                col16 = jnp.full((16,), c, jnp.int32)
                g = plsc.load_gather(rowsb, [row16, col16])
                stage_v[pl.ds(c * CH + v * 16, 16)] = g * mult
            return 0

        lax.fori_loop(0, CHV, tv, 0)

    def issue_out(k):
        col = seg_lo + k * CH
        for c in range(C):
            pltpu.async_copy(
                stage_v.at[pl.ds(c * CH, CH)],
                out_hbm.at[pl.ds((b * C + c) * NYNX + col, CH)],
                osem,
            )

    def wait_out(k):
        pltpu.make_async_copy(out_hbm.at[pl.ds(0, C * CH)], stage_v, osem).wait()

    prep_idx(0, idx0)
    issue_gather(idx0, rows0, gsem0)
    prep_idx(1, idx1)
    issue_gather(idx1, rows1, gsem1)

    def body(gg, _):
        k = 2 * gg
        wait_gather(idx0, rows0, gsem0)

        @pl.when(gg > 0)
        def _():
            wait_out(k - 1)

        transpose(k, rows0)
        prep_idx(k + 2, idx0)
        issue_gather(idx0, rows0, gsem0)
        issue_out(k)

        wait_gather(idx1, rows1, gsem1)
        wait_out(k)
        transpose(k + 1, rows1)

        @pl.when(gg < NCH // 2 - 1)
        def _():
            prep_idx(k + 3, idx1)
            issue_gather(idx1, rows1, gsem1)

        issue_out(k + 1)
        return 0

    lax.fori_loop(0, NCH // 2, body, 0)
    wait_gather(idx0, rows0, gsem0)
    wait_out(NCH - 2)
    transpose(NCH - 1, rows0)
    issue_out(NCH - 1)
    wait_out(NCH - 1)


def kernel(voxel_features, coors, batch_size):
    del batch_size
    flat = (coors[:, 0] * NYNX + coors[:, 2] * NX + coors[:, 3]).astype(jnp.int32)
    vfp = jnp.pad(voxel_features, ((0, 0), (0, CP - C)))
    inv = _build_inv(flat)
    out = _fill_canvas(vfp, inv)
    return out.reshape(BS, C, NY, NX)


# speedup vs baseline: 15.4988x; 15.4988x over previous
import functools

import jax
import jax.numpy as jnp
from jax import lax
from jax.experimental import pallas as pl
from jax.experimental.pallas import tpu as pltpu
from jax.experimental.pallas import tpu_sc as plsc

NY, NX, C, N, BS = 496, 432, 64, 40000, 4
NYNX = NY * NX
BASE = BS * NYNX
NTILES = 32
SEG = BASE // NTILES
SEG_V = SEG // 16
NHALF = N // 2
UF = 6

_MESH = plsc.VectorSubcoreMesh(core_axis_name="c", subcore_axis_name="s")
_PARAMS = pltpu.CompilerParams(needs_layout_passes=False)


@functools.partial(
    pl.pallas_call,
    out_shape=jax.ShapeDtypeStruct((C, N), jnp.float32),
)
def _transpose_tc(vf_ref, vft_ref):
    vft_ref[...] = vf_ref[...].T


@functools.partial(
    pl.kernel,
    out_type=jax.ShapeDtypeStruct((BS * C * NYNX,), jnp.float32),
    mesh=_MESH,
    compiler_params=_PARAMS,
    scratch_types=[
        pltpu.VMEM((SEG,), jnp.int32),
        pltpu.VMEM((N,), jnp.float32),
        pltpu.VMEM((SEG,), jnp.float32),
        pltpu.VMEM((NHALF,), jnp.int32),
    ],
)
def _fill_canvas(vft_hbm, flat_hbm, out_hbm, inv_v, col_v, stage_v, flat_v):
    wid = lax.axis_index("s") * 2 + lax.axis_index("c")
    b = wid // 8
    seg_lo = (wid % 8) * SEG
    lo = wid * SEG

    empty = jnp.full((16,), -1, jnp.int32)

    def fill(i, _):
        inv_v[pl.ds(i * 16, 16)] = empty
        return 0

    lax.fori_loop(0, SEG_V, fill, 0)

    lane = lax.iota(jnp.int32, 16)

    for half in (0, 1):
        pltpu.sync_copy(flat_hbm.at[pl.ds(half * NHALF, NHALF)], flat_v)

        def scan(i, _):
            base16 = flat_v[pl.ds(i * 16, 16)]
            loc = base16 - lo
            mask = (loc >= 0) & (loc < SEG)
            loc = jnp.where(mask, loc, 0)
            ids = lane + (i * 16 + half * NHALF)
            plsc.store_scatter(inv_v, [loc], ids, mask=mask)
            return 0

        lax.fori_loop(0, NHALF // 16, scan, 0)

    def chan(c, _):
        pltpu.sync_copy(vft_hbm.at[c], col_v)

        def gat(j, _):
            for u in range(UF):
                off = (j * UF + u) * 16
                iv = inv_v[pl.ds(off, 16)]
                idx = jnp.maximum(iv, 0)
                mult = jnp.where(iv >= 0, jnp.float32(1.0), jnp.float32(0.0))
                g = plsc.load_gather(col_v, [idx])
                stage_v[pl.ds(off, 16)] = g * mult
            return 0

        lax.fori_loop(0, SEG_V // UF, gat, 0)
        pltpu.sync_copy(stage_v,
                        out_hbm.at[pl.ds((b * C + c) * NYNX + seg_lo, SEG)])
        return 0

    lax.fori_loop(0, C, chan, 0)


def kernel(voxel_features, coors, batch_size):
    del batch_size
    flat = (coors[:, 0] * NYNX + coors[:, 2] * NX + coors[:, 3]).astype(jnp.int32)
    vft = _transpose_tc(voxel_features)
    out = _fill_canvas(vft, flat)
    return out.reshape(BS, C, NY, NX)


# speedup vs baseline: 15.6268x; 1.0083x over previous
import functools

import jax
import jax.numpy as jnp
from jax import lax
from jax.experimental import pallas as pl
from jax.experimental.pallas import tpu as pltpu
from jax.experimental.pallas import tpu_sc as plsc

NY, NX, C, N, BS = 496, 432, 64, 40000, 4
NYNX = NY * NX
BASE = BS * NYNX
NTILES = 32
SEG = BASE // NTILES
SEG_V = SEG // 16
NHALF = N // 2
UF = 6

_MESH = plsc.VectorSubcoreMesh(core_axis_name="c", subcore_axis_name="s")
_PARAMS = pltpu.CompilerParams(needs_layout_passes=False)


@functools.partial(
    pl.kernel,
    out_type=jax.ShapeDtypeStruct((BS * C * NYNX,), jnp.float32),
    mesh=_MESH,
    compiler_params=_PARAMS,
    scratch_types=[
        pltpu.VMEM((SEG,), jnp.int32),
        pltpu.VMEM((N,), jnp.float32),
        pltpu.VMEM((SEG,), jnp.float32),
        pltpu.VMEM((NHALF,), jnp.int32),
    ],
)
def _fill_canvas(vft_hbm, flat_hbm, out_hbm, inv_v, col_v, stage_v, flat_v):
    wid = lax.axis_index("s") * 2 + lax.axis_index("c")
    b = wid // 8
    seg_lo = (wid % 8) * SEG
    lo = wid * SEG

    empty = jnp.full((16,), -1, jnp.int32)

    def fill(i, _):
        inv_v[pl.ds(i * 16, 16)] = empty
        return 0

    lax.fori_loop(0, SEG_V, fill, 0)

    lane = lax.iota(jnp.int32, 16)

    for half in (0, 1):
        pltpu.sync_copy(flat_hbm.at[pl.ds(half * NHALF, NHALF)], flat_v)

        def scan(i, _):
            base16 = flat_v[pl.ds(i * 16, 16)]
            loc = base16 - lo
            mask = (loc >= 0) & (loc < SEG)
            loc = jnp.where(mask, loc, 0)
            ids = lane + (i * 16 + half * NHALF)
            plsc.store_scatter(inv_v, [loc], ids, mask=mask)
            return 0

        lax.fori_loop(0, NHALF // 16, scan, 0)

    def chan(c, _):
        pltpu.sync_copy(vft_hbm.at[pl.ds(c * N, N)], col_v)

        def gat(j, _):
            for u in range(UF):
                off = (j * UF + u) * 16
                iv = inv_v[pl.ds(off, 16)]
                idx = jnp.maximum(iv, 0)
                mult = jnp.where(iv >= 0, jnp.float32(1.0), jnp.float32(0.0))
                g = plsc.load_gather(col_v, [idx])
                stage_v[pl.ds(off, 16)] = g * mult
            return 0

        lax.fori_loop(0, SEG_V // UF, gat, 0)
        pltpu.sync_copy(stage_v,
                        out_hbm.at[pl.ds((b * C + c) * NYNX + seg_lo, SEG)])
        return 0

    lax.fori_loop(0, C, chan, 0)


def kernel(voxel_features, coors, batch_size):
    del batch_size
    flat = (coors[:, 0] * NYNX + coors[:, 2] * NX + coors[:, 3]).astype(jnp.int32)
    vft = voxel_features.T.reshape(-1)
    out = _fill_canvas(vft, flat)
    return out.reshape(BS, C, NY, NX)
